# baseline (speedup 1.0000x reference)
import functools

import jax
import jax.numpy as jnp
from jax import lax
from jax.experimental import pallas as pl
from jax.experimental.pallas import tpu as pltpu
from jax.experimental.pallas import tpu_sc as plsc

_N_NODES = 100000
_MEM_DIM = 128
_BATCH = 16384

_NC = 2
_NS = 16
_NW = _NC * _NS
_BPW = _BATCH // _NW
_CH = 128
_NCH = _BPW // _CH

def _wid():
  return lax.axis_index("s") * _NC + lax.axis_index("c")


_WCH = 2048
_NWCH = _BATCH // _WCH


@functools.cache
def _sc_kernels():
  mesh = plsc.VectorSubcoreMesh(core_axis_name="c", subcore_axis_name="s")

  @functools.partial(
      pl.kernel,
      out_type=jax.ShapeDtypeStruct((_N_NODES,), jnp.int32),
      mesh=mesh,
      compiler_params=pltpu.CompilerParams(needs_layout_passes=False),
      scratch_types=[
          pltpu.VMEM((_N_NODES,), jnp.int32),
          pltpu.VMEM((_WCH,), jnp.int32),
      ],
  )
  def sc_win(ids_hbm, slot_hbm, slot_v, ids_v):
    wid = _wid()

    @pl.when(wid == 0)
    def _():
      iota = lax.broadcasted_iota(jnp.int32, (16,), 0)
      perm = jnp.minimum(iota + 1, 15)

      @pl.loop(0, _NWCH)
      def _(c):
        pltpu.sync_copy(ids_hbm.at[pl.ds(c * _WCH, _WCH)], ids_v)

        @pl.loop(0, _WCH // 16, unroll=8)
        def _(v):
          ids16 = ids_v[pl.ds(v * 16, 16)]
          iar16 = iota + (c * _WCH + v * 16)
          key16 = ids16 * 16384 + iar16
          ks, _ = plsc.sort_key_val(key16, iar16)
          ids_s = lax.shift_right_logical(ks, 14)
          iar_s = jnp.bitwise_and(ks, 16383)
          ids_n = lax.gather(
              ids_s, perm[:, None],
              dimension_numbers=lax.GatherDimensionNumbers(
                  offset_dims=(), collapsed_slice_dims=(0,),
                  start_index_map=(0,)),
              slice_sizes=(1,),
              mode=lax.GatherScatterMode.PROMISE_IN_BOUNDS)
          m_last = (ids_s != ids_n) | (iota == 15)
          plsc.store_scatter(slot_v, [ids_s], iar_s, mask=m_last)

      pltpu.sync_copy(slot_v, slot_hbm)

  @functools.partial(
      pl.kernel,
      out_type=jax.ShapeDtypeStruct((_BATCH, _MEM_DIM), jnp.float32),
      mesh=mesh,
      scratch_types=[
          pltpu.VMEM((_NCH, _CH), jnp.int32),
          pltpu.VMEM((_BPW, _MEM_DIM), jnp.float32),
          pltpu.SemaphoreType.DMA,
      ],
  )
  def sc_gather(table_hbm, idx_hbm, out_hbm, idx_v, rows_v, sem):
    wid = _wid()
    rbase = wid * _NCH
    pltpu.sync_copy(idx_hbm.at[pl.ds(rbase, _NCH)], idx_v)
    cps = [
        pltpu.async_copy(table_hbm.at[idx_v.at[c]],
                         rows_v.at[pl.ds(c * _CH, _CH)], sem)
        for c in range(_NCH)
    ]
    for cp in cps:
      cp.wait()
    pltpu.sync_copy(rows_v, out_hbm.at[pl.ds(wid * _BPW, _BPW)])

  @functools.partial(
      pl.kernel,
      out_type=(),
      mesh=mesh,
      scratch_types=[
          pltpu.VMEM((_NCH, _CH), jnp.int32),
          pltpu.VMEM((_NCH, _CH), jnp.int32),
          pltpu.VMEM((_BPW, _MEM_DIM), jnp.float32),
          pltpu.VMEM((_NCH, _CH), jnp.float32),
          pltpu.SemaphoreType.DMA,
          pltpu.SemaphoreType.DMA,
          pltpu.SemaphoreType.DMA,
      ],
  )
  def sc_scatter(upd_hbm, idx_hbm, slot_hbm, ts_hbm, mem_ref, times_ref,
                 idx_v, win_v, rows_v, ts_v, sem_rows, sem_ts, sem_win):
    wid = _wid()
    rbase = wid * _NCH
    pltpu.sync_copy(idx_hbm.at[pl.ds(rbase, _NCH)], idx_v)
    wcps = [
        pltpu.async_copy(slot_hbm.at[idx_v.at[c]], win_v.at[c], sem_win)
        for c in range(_NCH)
    ]
    for cp in wcps:
      cp.wait()
    gcps, tcps = [], []
    for c in range(_NCH):
      gcps.append(
          pltpu.async_copy(upd_hbm.at[win_v.at[c]],
                           rows_v.at[pl.ds(c * _CH, _CH)], sem_rows))
      tcps.append(
          pltpu.async_copy(ts_hbm.at[win_v.at[c]], ts_v.at[c], sem_ts))
    for cp in gcps + tcps:
      cp.wait()
    scps = []
    for c in range(_NCH):
      scps.append(
          pltpu.async_copy(rows_v.at[pl.ds(c * _CH, _CH)],
                           mem_ref.at[idx_v.at[c]], sem_rows))
      scps.append(
          pltpu.async_copy(ts_v.at[c], times_ref.at[idx_v.at[c]], sem_ts))
    for cp in scps:
      cp.wait()

  return sc_win, sc_gather, sc_scatter


_GRU_BLK = 1024


def _gru_body(x_ref, h_ref, wx_ref, wh_ref, bx_ref, bh_ref, o_ref):
  x = x_ref[...]
  h = h_ref[...]
  gx = jnp.dot(x, wx_ref[...], preferred_element_type=jnp.float32) + bx_ref[...]
  gh = jnp.dot(h, wh_ref[...], preferred_element_type=jnp.float32) + bh_ref[...]
  d = _MEM_DIM
  r = jax.nn.sigmoid(gx[:, :d] + gh[:, :d])
  z = jax.nn.sigmoid(gx[:, d:2 * d] + gh[:, d:2 * d])
  n = jnp.tanh(gx[:, 2 * d:] + r * gh[:, 2 * d:])
  o_ref[...] = (1.0 - z) * n + z * h


_tc_gru = pl.pallas_call(
    _gru_body,
    grid=(_BATCH // _GRU_BLK,),
    in_specs=[
        pl.BlockSpec((_GRU_BLK, _MEM_DIM), lambda i: (i, 0)),
        pl.BlockSpec((_GRU_BLK, _MEM_DIM), lambda i: (i, 0)),
        pl.BlockSpec((_MEM_DIM, 3 * _MEM_DIM), lambda i: (0, 0)),
        pl.BlockSpec((_MEM_DIM, 3 * _MEM_DIM), lambda i: (0, 0)),
        pl.BlockSpec((1, 3 * _MEM_DIM), lambda i: (0, 0)),
        pl.BlockSpec((1, 3 * _MEM_DIM), lambda i: (0, 0)),
    ],
    out_specs=pl.BlockSpec((_GRU_BLK, _MEM_DIM), lambda i: (i, 0)),
    out_shape=jax.ShapeDtypeStruct((_BATCH, _MEM_DIM), jnp.float32),
)


def kernel(node_memories, node_last_updated_times, unique_node_ids,
           unique_node_messages, unique_node_timestamps, W_x, W_h, b_x, b_h):
  ids = unique_node_ids.astype(jnp.int32)
  ids2 = ids.reshape(_NW * _NCH, _CH)

  sc_win, sc_gather, sc_scatter = _sc_kernels()
  mem_ref = jax.new_ref(node_memories)
  times_ref = jax.new_ref(node_last_updated_times)
  gathered = sc_gather(node_memories, ids2)
  slot = sc_win(ids)
  upd = _tc_gru(unique_node_messages, gathered, W_x, W_h,
                b_x.reshape(1, -1), b_h.reshape(1, -1))

  sc_scatter(upd, ids2, slot, unique_node_timestamps, mem_ref, times_ref)
  return mem_ref[...], times_ref[...]


# speedup vs baseline: 1.0490x; 1.0490x over previous
import functools

import jax
import jax.numpy as jnp
from jax import lax
from jax.experimental import pallas as pl
from jax.experimental.pallas import tpu as pltpu
from jax.experimental.pallas import tpu_sc as plsc

_N_NODES = 100000
_MEM_DIM = 128
_BATCH = 16384

_NC = 2
_NS = 16
_NW = _NC * _NS
_BPW = _BATCH // _NW
_CH = 128
_NCH = _BPW // _CH

def _wid():
  return lax.axis_index("s") * _NC + lax.axis_index("c")


_WCH = 2048
_NWCH = _BATCH // _WCH

_CPR = 192
_NFULL = _N_NODES // _CPR
_REM = _N_NODES - _NFULL * _CPR


@functools.cache
def _sc_kernels():
  mesh = plsc.VectorSubcoreMesh(core_axis_name="c", subcore_axis_name="s")

  @functools.partial(
      pl.kernel,
      out_type=jax.ShapeDtypeStruct((_N_NODES,), jnp.int32),
      mesh=mesh,
      compiler_params=pltpu.CompilerParams(needs_layout_passes=False),
      scratch_types=[
          pltpu.VMEM((_N_NODES,), jnp.int32),
          pltpu.VMEM((_WCH,), jnp.int32),
          pltpu.VMEM((_CPR, _MEM_DIM), jnp.float32),
      ],
  )
  def sc_win(ids_hbm, table_hbm, mem_ref, slot_hbm, slot_v, ids_v, copy_v):
    wid = _wid()

    @pl.when(wid != 0)
    def _():
      w = wid - 1

      @pl.loop(w, _NFULL, step=31)
      def _(c):
        pltpu.sync_copy(table_hbm.at[pl.ds(c * _CPR, _CPR)], copy_v)
        pltpu.sync_copy(copy_v, mem_ref.at[pl.ds(c * _CPR, _CPR)])

      @pl.when(w == 0)
      def _():
        pltpu.sync_copy(table_hbm.at[pl.ds(_NFULL * _CPR, _REM)],
                        copy_v.at[pl.ds(0, _REM)])
        pltpu.sync_copy(copy_v.at[pl.ds(0, _REM)],
                        mem_ref.at[pl.ds(_NFULL * _CPR, _REM)])

    @pl.when(wid == 0)
    def _():
      iota = lax.broadcasted_iota(jnp.int32, (16,), 0)
      perm = jnp.minimum(iota + 1, 15)

      @pl.loop(0, _NWCH)
      def _(c):
        pltpu.sync_copy(ids_hbm.at[pl.ds(c * _WCH, _WCH)], ids_v)

        @pl.loop(0, _WCH // 16, unroll=8)
        def _(v):
          ids16 = ids_v[pl.ds(v * 16, 16)]
          iar16 = iota + (c * _WCH + v * 16)
          key16 = ids16 * 16384 + iar16
          ks, _ = plsc.sort_key_val(key16, iar16)
          ids_s = lax.shift_right_logical(ks, 14)
          iar_s = jnp.bitwise_and(ks, 16383)
          ids_n = lax.gather(
              ids_s, perm[:, None],
              dimension_numbers=lax.GatherDimensionNumbers(
                  offset_dims=(), collapsed_slice_dims=(0,),
                  start_index_map=(0,)),
              slice_sizes=(1,),
              mode=lax.GatherScatterMode.PROMISE_IN_BOUNDS)
          m_last = (ids_s != ids_n) | (iota == 15)
          plsc.store_scatter(slot_v, [ids_s], iar_s, mask=m_last)

      pltpu.sync_copy(slot_v, slot_hbm)

  @functools.partial(
      pl.kernel,
      out_type=jax.ShapeDtypeStruct((_BATCH, _MEM_DIM), jnp.float32),
      mesh=mesh,
      scratch_types=[
          pltpu.VMEM((_NCH, _CH), jnp.int32),
          pltpu.VMEM((_BPW, _MEM_DIM), jnp.float32),
          pltpu.SemaphoreType.DMA,
      ],
  )
  def sc_gather(table_hbm, idx_hbm, out_hbm, idx_v, rows_v, sem):
    wid = _wid()
    rbase = wid * _NCH
    pltpu.sync_copy(idx_hbm.at[pl.ds(rbase, _NCH)], idx_v)
    cps = [
        pltpu.async_copy(table_hbm.at[idx_v.at[c]],
                         rows_v.at[pl.ds(c * _CH, _CH)], sem)
        for c in range(_NCH)
    ]
    for cp in cps:
      cp.wait()
    pltpu.sync_copy(rows_v, out_hbm.at[pl.ds(wid * _BPW, _BPW)])

  @functools.partial(
      pl.kernel,
      out_type=(),
      mesh=mesh,
      scratch_types=[
          pltpu.VMEM((_NCH, _CH), jnp.int32),
          pltpu.VMEM((_NCH, _CH), jnp.int32),
          pltpu.VMEM((_BPW, _MEM_DIM), jnp.float32),
          pltpu.VMEM((_NCH, _CH), jnp.float32),
          pltpu.SemaphoreType.DMA,
          pltpu.SemaphoreType.DMA,
          pltpu.SemaphoreType.DMA,
      ],
  )
  def sc_scatter(upd_hbm, idx_hbm, slot_hbm, ts_hbm, mem_ref, times_ref,
                 idx_v, win_v, rows_v, ts_v, sem_rows, sem_ts, sem_win):
    wid = _wid()
    rbase = wid * _NCH
    pltpu.sync_copy(idx_hbm.at[pl.ds(rbase, _NCH)], idx_v)
    wcps = [
        pltpu.async_copy(slot_hbm.at[idx_v.at[c]], win_v.at[c], sem_win)
        for c in range(_NCH)
    ]
    for cp in wcps:
      cp.wait()
    gcps, tcps = [], []
    for c in range(_NCH):
      gcps.append(
          pltpu.async_copy(upd_hbm.at[win_v.at[c]],
                           rows_v.at[pl.ds(c * _CH, _CH)], sem_rows))
      tcps.append(
          pltpu.async_copy(ts_hbm.at[win_v.at[c]], ts_v.at[c], sem_ts))
    for cp in gcps + tcps:
      cp.wait()
    scps = []
    for c in range(_NCH):
      scps.append(
          pltpu.async_copy(rows_v.at[pl.ds(c * _CH, _CH)],
                           mem_ref.at[idx_v.at[c]], sem_rows))
      scps.append(
          pltpu.async_copy(ts_v.at[c], times_ref.at[idx_v.at[c]], sem_ts))
    for cp in scps:
      cp.wait()

  return sc_win, sc_gather, sc_scatter


_GRU_BLK = 1024


def _gru_body(x_ref, h_ref, wx_ref, wh_ref, bx_ref, bh_ref, o_ref):
  x = x_ref[...]
  h = h_ref[...]
  gx = jnp.dot(x, wx_ref[...], preferred_element_type=jnp.float32) + bx_ref[...]
  gh = jnp.dot(h, wh_ref[...], preferred_element_type=jnp.float32) + bh_ref[...]
  d = _MEM_DIM
  r = jax.nn.sigmoid(gx[:, :d] + gh[:, :d])
  z = jax.nn.sigmoid(gx[:, d:2 * d] + gh[:, d:2 * d])
  n = jnp.tanh(gx[:, 2 * d:] + r * gh[:, 2 * d:])
  o_ref[...] = (1.0 - z) * n + z * h


_tc_gru = pl.pallas_call(
    _gru_body,
    grid=(_BATCH // _GRU_BLK,),
    in_specs=[
        pl.BlockSpec((_GRU_BLK, _MEM_DIM), lambda i: (i, 0)),
        pl.BlockSpec((_GRU_BLK, _MEM_DIM), lambda i: (i, 0)),
        pl.BlockSpec((_MEM_DIM, 3 * _MEM_DIM), lambda i: (0, 0)),
        pl.BlockSpec((_MEM_DIM, 3 * _MEM_DIM), lambda i: (0, 0)),
        pl.BlockSpec((1, 3 * _MEM_DIM), lambda i: (0, 0)),
        pl.BlockSpec((1, 3 * _MEM_DIM), lambda i: (0, 0)),
    ],
    out_specs=pl.BlockSpec((_GRU_BLK, _MEM_DIM), lambda i: (i, 0)),
    out_shape=jax.ShapeDtypeStruct((_BATCH, _MEM_DIM), jnp.float32),
)


def kernel(node_memories, node_last_updated_times, unique_node_ids,
           unique_node_messages, unique_node_timestamps, W_x, W_h, b_x, b_h):
  ids = unique_node_ids.astype(jnp.int32)
  ids2 = ids.reshape(_NW * _NCH, _CH)

  sc_win, sc_gather, sc_scatter = _sc_kernels()
  mem_ref = jax.empty_ref(
      jax.ShapeDtypeStruct((_N_NODES, _MEM_DIM), jnp.float32))
  times_ref = jax.new_ref(node_last_updated_times)
  gathered = sc_gather(node_memories, ids2)
  slot = sc_win(ids, node_memories, mem_ref)
  upd = _tc_gru(unique_node_messages, gathered, W_x, W_h,
                b_x.reshape(1, -1), b_h.reshape(1, -1))

  sc_scatter(upd, ids2, slot, unique_node_timestamps, mem_ref, times_ref)
  return mem_ref[...], times_ref[...]
